# speedup vs baseline: 3.3373x; 3.3373x over previous
import jax
import jax.numpy as jnp
from jax import lax
from jax.experimental import pallas as pl
from jax.experimental.pallas import tpu as pltpu
from jax.experimental.pallas import tpu_sc as plsc

NC = 2
NS = 16
NW = NC * NS

D = 128
B_TOT = 4096 * 50
BPW = B_TOT // NW
C = 400
NCH = BPW // C


def _emb_body(idx_hbm, w_hbm, out_hbm, idx_v, rows0, rows1, gsem0, gsem1,
              ssem0, ssem1):
    wid = lax.axis_index("s") * NC + lax.axis_index("c")
    base = wid * BPW
    pltpu.sync_copy(idx_hbm.at[pl.ds(base, BPW)], idx_v)

    rows = (rows0, rows1)
    gsem = (gsem0, gsem1)
    ssem = (ssem0, ssem1)
    gdesc = [None, None]
    sdesc = [None, None]
    for i in range(NCH):
        b = i & 1
        if i >= 2:
            sdesc[b].wait()
        gdesc[b] = pltpu.async_copy(
            w_hbm.at[idx_v.at[pl.ds(i * C, C)]], rows[b], gsem[b])
        if i >= 1:
            pb = (i - 1) & 1
            gdesc[pb].wait()
            sdesc[pb] = pltpu.async_copy(
                rows[pb], out_hbm.at[pl.ds(base + (i - 1) * C, C)], ssem[pb])
    last = (NCH - 1) & 1
    gdesc[last].wait()
    sdesc[last] = pltpu.async_copy(
        rows[last], out_hbm.at[pl.ds(base + (NCH - 1) * C, C)], ssem[last])
    sdesc[(NCH - 2) & 1].wait()
    sdesc[last].wait()


def kernel(indices, weight):
    idx = indices.reshape(-1).astype(jnp.int32)
    mesh = plsc.VectorSubcoreMesh(
        core_axis_name="c", subcore_axis_name="s",
        num_cores=NC, num_subcores=NS)
    out = pl.kernel(
        _emb_body,
        out_type=jax.ShapeDtypeStruct((B_TOT, D), jnp.float32),
        mesh=mesh,
        scratch_types=[
            pltpu.VMEM((BPW,), jnp.int32),
            pltpu.VMEM((C, D), jnp.float32),
            pltpu.VMEM((C, D), jnp.float32),
            pltpu.SemaphoreType.DMA,
            pltpu.SemaphoreType.DMA,
            pltpu.SemaphoreType.DMA,
            pltpu.SemaphoreType.DMA,
        ],
    )(idx, weight)
    return out.reshape(indices.shape + (D,))


# speedup vs baseline: 5.8777x; 1.7612x over previous
import jax
import jax.numpy as jnp
from jax import lax
from jax.experimental import pallas as pl
from jax.experimental.pallas import tpu as pltpu
from jax.experimental.pallas import tpu_sc as plsc

NC = 2
NS = 16
NW = NC * NS

D = 128
B_TOT = 4096 * 50
BPW = B_TOT // NW
C = 400
NCH = BPW // C


SEQ = 50
BATCH = 4096
BAT_PER_W = BATCH // NW
BAT_PER_CH = C // SEQ


def _emb_body(idx_hbm, w_hbm, out_hbm, idx_v, rows0, rows1, gsem0, gsem1,
              ssem0, ssem1):
    wid = lax.axis_index("s") * NC + lax.axis_index("c")
    base = wid * BPW
    bbase = wid * BAT_PER_W
    pltpu.sync_copy(idx_hbm.at[pl.ds(base, BPW)], idx_v)

    rows = (rows0, rows1)
    gsem = (gsem0, gsem1)
    ssem = (ssem0, ssem1)
    gdesc = [None, None]
    sdesc = [None, None]
    for i in range(NCH):
        b = i & 1
        if i >= 2:
            sdesc[b].wait()
        gdesc[b] = pltpu.async_copy(
            w_hbm.at[idx_v.at[pl.ds(i * C, C)]], rows[b], gsem[b])
        if i >= 1:
            pb = (i - 1) & 1
            gdesc[pb].wait()
            sdesc[pb] = pltpu.async_copy(
                rows[pb].reshape(BAT_PER_CH, SEQ, D),
                out_hbm.at[pl.ds(bbase + (i - 1) * BAT_PER_CH, BAT_PER_CH)],
                ssem[pb])
    last = (NCH - 1) & 1
    gdesc[last].wait()
    sdesc[last] = pltpu.async_copy(
        rows[last].reshape(BAT_PER_CH, SEQ, D),
        out_hbm.at[pl.ds(bbase + (NCH - 1) * BAT_PER_CH, BAT_PER_CH)],
        ssem[last])
    sdesc[(NCH - 2) & 1].wait()
    sdesc[last].wait()


def kernel(indices, weight):
    idx = indices.reshape(-1).astype(jnp.int32)
    mesh = plsc.VectorSubcoreMesh(
        core_axis_name="c", subcore_axis_name="s",
        num_cores=NC, num_subcores=NS)
    out = pl.kernel(
        _emb_body,
        out_type=jax.ShapeDtypeStruct((BATCH, SEQ, D), jnp.float32),
        mesh=mesh,
        scratch_types=[
            pltpu.VMEM((BPW,), jnp.int32),
            pltpu.VMEM((C, D), jnp.float32),
            pltpu.VMEM((C, D), jnp.float32),
            pltpu.SemaphoreType.DMA,
            pltpu.SemaphoreType.DMA,
            pltpu.SemaphoreType.DMA,
            pltpu.SemaphoreType.DMA,
        ],
    )(idx, weight)
    return out


# speedup vs baseline: 5.8965x; 1.0032x over previous
import jax
import jax.numpy as jnp
from jax import lax
from jax.experimental import pallas as pl
from jax.experimental.pallas import tpu as pltpu
from jax.experimental.pallas import tpu_sc as plsc

NC = 2
NS = 16
NW = NC * NS

D = 128
B_TOT = 4096 * 50
BPW = B_TOT // NW
C = 400
NCH = BPW // C


SEQ = 50
BATCH = 4096
BAT_PER_W = BATCH // NW
BAT_PER_CH = C // SEQ


def _emb_body(idx_hbm, w_hbm, out_hbm, idx_v, rows0, rows1, gsem0, gsem1,
              ssem0, ssem1):
    wid = lax.axis_index("s") * NC + lax.axis_index("c")
    base = wid * BPW
    bbase = wid * BAT_PER_W
    pltpu.sync_copy(idx_hbm.at[pl.ds(base, BPW)], idx_v)

    rows = (rows0, rows1)
    gsem = (gsem0, gsem1)
    ssem = (ssem0, ssem1)
    gdesc = [None, None]
    sdesc = [None, None]
    for i in range(NCH):
        b = i & 1
        if i >= 2:
            sdesc[b].wait()
        gdesc[b] = pltpu.async_copy(
            w_hbm.at[idx_v.at[pl.ds(i * C, C)]], rows[b], gsem[b])
        if i >= 1:
            pb = (i - 1) & 1
            gdesc[pb].wait()
            sdesc[pb] = pltpu.async_copy(
                rows[pb].reshape(BAT_PER_CH, SEQ, D),
                out_hbm.at[pl.ds(bbase + (i - 1) * BAT_PER_CH, BAT_PER_CH)],
                ssem[pb])
    last = (NCH - 1) & 1
    gdesc[last].wait()
    sdesc[last] = pltpu.async_copy(
        rows[last].reshape(BAT_PER_CH, SEQ, D),
        out_hbm.at[pl.ds(bbase + (NCH - 1) * BAT_PER_CH, BAT_PER_CH)],
        ssem[last])
    sdesc[(NCH - 2) & 1].wait()
    sdesc[last].wait()


def kernel(indices, weight):
    idx = indices.reshape(-1).astype(jnp.int32)
    mesh = plsc.VectorSubcoreMesh(
        core_axis_name="c", subcore_axis_name="s",
        num_cores=NC, num_subcores=NS)
    out = pl.kernel(
        _emb_body,
        out_type=jax.ShapeDtypeStruct((BATCH, SEQ, D), jnp.float32),
        mesh=mesh,
        compiler_params=pltpu.CompilerParams(use_tc_tiling_on_sc=True),
        scratch_types=[
            pltpu.VMEM((BPW,), jnp.int32),
            pltpu.VMEM((C, D), jnp.float32),
            pltpu.VMEM((C, D), jnp.float32),
            pltpu.SemaphoreType.DMA,
            pltpu.SemaphoreType.DMA,
            pltpu.SemaphoreType.DMA,
            pltpu.SemaphoreType.DMA,
        ],
    )(idx, weight)
    return out


# speedup vs baseline: 10.4020x; 1.7641x over previous
import jax
import jax.numpy as jnp
from jax import lax
from jax.experimental import pallas as pl
from jax.experimental.pallas import tpu as pltpu
from jax.experimental.pallas import tpu_sc as plsc

NC = 2
NS = 16
NW = NC * NS

D = 128
SEQ = 50
BATCH = 4096
B_TOT = BATCH * SEQ
BPW = B_TOT // NW
C = 400
NCH = BPW // C


def _emb_body(idx_hbm, w_hbm, out_hbm, idx_v, rows0, rows1, gsem0, gsem1,
              ssem0, ssem1):
    wid = lax.axis_index("s") * NC + lax.axis_index("c")
    base = wid * BPW
    pltpu.sync_copy(idx_hbm.at[pl.ds(base, BPW)], idx_v)

    rows = (rows0, rows1)
    gsem = (gsem0, gsem1)
    ssem = (ssem0, ssem1)
    gdesc = [None, None]
    sdesc = [None, None]
    for i in range(NCH):
        b = i & 1
        if i >= 2:
            sdesc[b].wait()
        gdesc[b] = pltpu.async_copy(
            w_hbm.at[idx_v.at[pl.ds(i * C, C)]], rows[b], gsem[b])
        if i >= 1:
            pb = (i - 1) & 1
            gdesc[pb].wait()
            sdesc[pb] = pltpu.async_copy(
                rows[pb], out_hbm.at[pl.ds(base + (i - 1) * C, C)], ssem[pb])
    last = (NCH - 1) & 1
    gdesc[last].wait()
    sdesc[last] = pltpu.async_copy(
        rows[last], out_hbm.at[pl.ds(base + (NCH - 1) * C, C)], ssem[last])
    sdesc[(NCH - 2) & 1].wait()
    sdesc[last].wait()


def kernel(indices, weight):
    idx_t = indices.T.reshape(-1).astype(jnp.int32)
    mesh = plsc.VectorSubcoreMesh(
        core_axis_name="c", subcore_axis_name="s",
        num_cores=NC, num_subcores=NS)
    out = pl.kernel(
        _emb_body,
        out_type=jax.ShapeDtypeStruct((B_TOT, D), jnp.float32),
        mesh=mesh,
        scratch_types=[
            pltpu.VMEM((BPW,), jnp.int32),
            pltpu.VMEM((C, D), jnp.float32),
            pltpu.VMEM((C, D), jnp.float32),
            pltpu.SemaphoreType.DMA,
            pltpu.SemaphoreType.DMA,
            pltpu.SemaphoreType.DMA,
            pltpu.SemaphoreType.DMA,
        ],
    )(idx_t, weight)
    return jnp.swapaxes(out.reshape(SEQ, BATCH, D), 0, 1)
